# baseline (speedup 1.0000x reference)
import functools

import jax
import jax.numpy as jnp
from jax import lax
from jax.experimental import pallas as pl
from jax.experimental.pallas import tpu as pltpu
from jax.experimental.pallas import tpu_sc as plsc

VOCAB = 100000
EMBED_DIM = 16
BATCH = 1024

V_TILE = 4096
NV = (VOCAB + V_TILE - 1) // V_TILE
V_PAD = NV * V_TILE


@functools.lru_cache(maxsize=1)
def _make_sc_gather():
    info = plsc.get_sparse_core_info()
    nw = info.num_cores * info.num_subcores
    b_per_w = BATCH // nw
    mesh = plsc.VectorSubcoreMesh(core_axis_name="c", subcore_axis_name="s")

    @functools.partial(
        pl.kernel,
        mesh=mesh,
        out_type=jax.ShapeDtypeStruct((BATCH, 128), jnp.float32),
        scratch_types=[
            pltpu.VMEM((b_per_w,), jnp.int32),
            pltpu.VMEM((b_per_w, EMBED_DIM), jnp.float32),
            pltpu.SemaphoreType.DMA,
        ],
        compiler_params=pltpu.CompilerParams(use_tc_tiling_on_sc=False),
    )
    def gather(table_hbm, idx_hbm, out_hbm, idx_v, rows_v, sem):
        wid = lax.axis_index("s") * info.num_cores + lax.axis_index("c")
        base = wid * b_per_w
        pltpu.sync_copy(idx_hbm.at[pl.ds(base, b_per_w)], idx_v)
        pltpu.async_copy(table_hbm.at[idx_v], rows_v, sem).wait()
        pltpu.sync_copy(rows_v, out_hbm.at[pl.ds(base, b_per_w), pl.ds(0, EMBED_DIM)])

    return gather


_LN2 = 0.6931471805599453


def _stats_body(e_ref, w_ref, b_ref, lse_ref, s_ref):
    v = pl.program_id(0)

    @pl.when(v == 0)
    def _init():
        s_ref[...] = jnp.zeros_like(s_ref)

    l2 = lax.dot_general(
        e_ref[...][:, :EMBED_DIM], w_ref[...], (((1,), (0,)), ((), ())),
        preferred_element_type=jnp.float32,
    ) + b_ref[...]
    p = jnp.exp2(l2)

    acc = s_ref[...]
    for i in range(V_TILE // 128):
        acc = acc + p[:, i * 128:(i + 1) * 128]
    s_ref[...] = acc

    @pl.when(v == NV - 1)
    def _fin():
        lse_ref[...] = jnp.log2(jnp.sum(s_ref[...], axis=1, keepdims=True))


def _write_body(e_ref, w_ref, b_ref, lse_ref, o_ref):
    l2 = lax.dot_general(
        e_ref[...][:, :EMBED_DIM], w_ref[...], (((1,), (0,)), ((), ())),
        preferred_element_type=jnp.float32,
    ) + b_ref[...]
    o_ref[...] = (l2 - lse_ref[...]) * _LN2


def kernel(inputs, emb_table, W, b):
    embeds = _make_sc_gather()(emb_table, inputs.astype(jnp.int32))

    log2e = jnp.float32(1.4426950408889634)
    W_pad = jnp.pad(W.T * log2e, ((0, 0), (0, V_PAD - VOCAB)))
    b_pad = jnp.pad((b * log2e).reshape(1, VOCAB),
                    ((0, 0), (0, V_PAD - VOCAB)), constant_values=-1e30)

    lse = pl.pallas_call(
        _stats_body,
        grid=(NV,),
        in_specs=[
            pl.BlockSpec((BATCH, 128), lambda v: (0, 0)),
            pl.BlockSpec((EMBED_DIM, V_TILE), lambda v: (0, v)),
            pl.BlockSpec((1, V_TILE), lambda v: (0, v)),
        ],
        out_specs=pl.BlockSpec((BATCH, 1), lambda v: (0, 0)),
        out_shape=jax.ShapeDtypeStruct((BATCH, 1), jnp.float32),
        scratch_shapes=[
            pltpu.VMEM((BATCH, 128), jnp.float32),
        ],
    )(embeds, W_pad, b_pad)

    log_probs = pl.pallas_call(
        _write_body,
        grid=(NV,),
        in_specs=[
            pl.BlockSpec((BATCH, 128), lambda v: (0, 0)),
            pl.BlockSpec((EMBED_DIM, V_TILE), lambda v: (0, v)),
            pl.BlockSpec((1, V_TILE), lambda v: (0, v)),
            pl.BlockSpec((BATCH, 1), lambda v: (0, 0)),
        ],
        out_specs=pl.BlockSpec((BATCH, V_TILE), lambda v: (0, v)),
        out_shape=jax.ShapeDtypeStruct((BATCH, VOCAB), jnp.float32),
    )(embeds, W_pad, b_pad, lse)

    return log_probs


# speedup vs baseline: 1.0018x; 1.0018x over previous
import functools

import jax
import jax.numpy as jnp
from jax import lax
from jax.experimental import pallas as pl
from jax.experimental.pallas import tpu as pltpu
from jax.experimental.pallas import tpu_sc as plsc

VOCAB = 100000
EMBED_DIM = 16
BATCH = 1024

V_TILE = 4096
NV = (VOCAB + V_TILE - 1) // V_TILE
S_TILE = 8192
NS = 13
V_PAD = NS * S_TILE


@functools.lru_cache(maxsize=1)
def _make_sc_gather():
    info = plsc.get_sparse_core_info()
    nw = info.num_cores * info.num_subcores
    b_per_w = BATCH // nw
    mesh = plsc.VectorSubcoreMesh(core_axis_name="c", subcore_axis_name="s")

    @functools.partial(
        pl.kernel,
        mesh=mesh,
        out_type=jax.ShapeDtypeStruct((BATCH, 128), jnp.float32),
        scratch_types=[
            pltpu.VMEM((b_per_w,), jnp.int32),
            pltpu.VMEM((b_per_w, EMBED_DIM), jnp.float32),
            pltpu.SemaphoreType.DMA,
        ],
        compiler_params=pltpu.CompilerParams(use_tc_tiling_on_sc=False),
    )
    def gather(table_hbm, idx_hbm, out_hbm, idx_v, rows_v, sem):
        wid = lax.axis_index("s") * info.num_cores + lax.axis_index("c")
        base = wid * b_per_w
        pltpu.sync_copy(idx_hbm.at[pl.ds(base, b_per_w)], idx_v)
        pltpu.async_copy(table_hbm.at[idx_v], rows_v, sem).wait()
        pltpu.sync_copy(rows_v, out_hbm.at[pl.ds(base, b_per_w), pl.ds(0, EMBED_DIM)])

    return gather


_LN2 = 0.6931471805599453


def _stats_body(e_ref, w_ref, b_ref, lse_ref, s_ref):
    v = pl.program_id(0)

    @pl.when(v == 0)
    def _init():
        s_ref[...] = jnp.zeros_like(s_ref)

    l2 = lax.dot_general(
        e_ref[...][:, :EMBED_DIM], w_ref[...], (((1,), (0,)), ((), ())),
        preferred_element_type=jnp.float32,
    ) + b_ref[...]
    p = jnp.exp2(l2)

    acc = s_ref[...]
    for i in range(S_TILE // 128):
        acc = acc + p[:, i * 128:(i + 1) * 128]
    s_ref[...] = acc

    @pl.when(v == NS - 1)
    def _fin():
        lse_ref[...] = jnp.log2(jnp.sum(s_ref[...], axis=1, keepdims=True))


def _write_body(e_ref, w_ref, b_ref, lse_ref, o_ref):
    l2 = lax.dot_general(
        e_ref[...][:, :EMBED_DIM], w_ref[...], (((1,), (0,)), ((), ())),
        preferred_element_type=jnp.float32,
    ) + b_ref[...]
    o_ref[...] = (l2 - lse_ref[...]) * _LN2


def kernel(inputs, emb_table, W, b):
    embeds = _make_sc_gather()(emb_table, inputs.astype(jnp.int32))

    log2e = jnp.float32(1.4426950408889634)
    W_pad = jnp.pad(W.T * log2e, ((0, 0), (0, V_PAD - VOCAB)))
    b_pad = jnp.pad((b * log2e).reshape(1, VOCAB),
                    ((0, 0), (0, V_PAD - VOCAB)), constant_values=-1e30)

    lse = pl.pallas_call(
        _stats_body,
        grid=(NS,),
        in_specs=[
            pl.BlockSpec((BATCH, 128), lambda v: (0, 0)),
            pl.BlockSpec((EMBED_DIM, S_TILE), lambda v: (0, v)),
            pl.BlockSpec((1, S_TILE), lambda v: (0, v)),
        ],
        out_specs=pl.BlockSpec((BATCH, 1), lambda v: (0, 0)),
        out_shape=jax.ShapeDtypeStruct((BATCH, 1), jnp.float32),
        scratch_shapes=[
            pltpu.VMEM((BATCH, 128), jnp.float32),
        ],
    )(embeds, W_pad, b_pad)

    log_probs = pl.pallas_call(
        _write_body,
        grid=(NV,),
        in_specs=[
            pl.BlockSpec((BATCH, 128), lambda v: (0, 0)),
            pl.BlockSpec((EMBED_DIM, V_TILE), lambda v: (0, v)),
            pl.BlockSpec((1, V_TILE), lambda v: (0, v)),
            pl.BlockSpec((BATCH, 1), lambda v: (0, 0)),
        ],
        out_specs=pl.BlockSpec((BATCH, V_TILE), lambda v: (0, v)),
        out_shape=jax.ShapeDtypeStruct((BATCH, VOCAB), jnp.float32),
    )(embeds, W_pad, b_pad, lse)

    return log_probs
